# baseline (speedup 1.0000x reference)
import functools
import math

import jax
import jax.numpy as jnp
from jax.experimental import pallas as pl
from jax.experimental.pallas import tpu as pltpu

_L = 12
_D = 768
_H = 12
_DH = 64
_FF = 3072
_B = 8
_S = 512
_M = _B * _S
_T = 256
_NT = _M // _T
_GC = 32


def _gather_index(j, i, idx):
    return (idx[i * _GC + j], 0, 0)


def _gather_kernel(idx_ref, *refs):
    del idx_ref
    out_ref = refs[-1]
    for j in range(_GC):
        out_ref[j : j + 1, :] = refs[j][0].astype(jnp.bfloat16)


def _ln(x, g, b):
    m = jnp.mean(x, axis=-1, keepdims=True)
    v = jnp.mean((x - m) ** 2, axis=-1, keepdims=True)
    return (x - m) * jax.lax.rsqrt(v + 1e-12) * g + b


def _mm(x, w):
    return jax.lax.dot_general(
        x, w, (((1,), (0,)), ((), ())), preferred_element_type=jnp.float32
    )


def _bmm(x, w):
    return jax.lax.dot_general(
        x.astype(jnp.bfloat16), w, (((1,), (0,)), ((), ())),
        preferred_element_type=jnp.float32,
    )


def _gelu(x):
    return 0.5 * x * (1.0 + jax.lax.erf(x * jnp.float32(1.0 / math.sqrt(2.0))))


def _mega_kernel(
    eraw_ref, pos_ref, tok_ref, eg_ref, eb_ref,
    wq_ref, bq_ref, wk_ref, bk_ref, wv_ref, bv_ref,
    wo_ref, bo_ref, l1g_ref, l1b_ref,
    w1_ref, b1_ref, w2_ref, b2_ref, l2g_ref, l2b_ref,
    out_ref, h_scr, kv_scr,
):
    l = pl.program_id(0)
    t = pl.program_id(1)
    b = t // 2
    half = t % 2
    seq0 = b * _S
    rows_seq = pl.ds(seq0, _S)
    rows_q = pl.ds(seq0 + half * _T, _T)

    @pl.when(jnp.logical_and(l == 0, half == 0))
    def _embed():
        pe = pos_ref[...].astype(jnp.float32) + tok_ref[...]
        e1 = _ln(eraw_ref[...].astype(jnp.float32) + pe, eg_ref[...], eb_ref[...])
        h_scr[rows_seq, :] = _ln(e1 + pe, eg_ref[...], eb_ref[...])

    @pl.when(half == 0)
    def _kv():
        xs = h_scr[rows_seq, :]
        kv_scr[:, :_D] = (_bmm(xs, wk_ref[0]) + bk_ref[0]).astype(jnp.bfloat16)
        kv_scr[:, _D:] = (_bmm(xs, wv_ref[0]) + bv_ref[0]).astype(jnp.bfloat16)

    x = h_scr[rows_q, :]
    q = _bmm(x, wq_ref[0]) + bq_ref[0]
    scale = jnp.float32(1.0 / math.sqrt(_DH))
    pieces = []
    for hh in range(_H):
        sl = slice(hh * _DH, (hh + 1) * _DH)
        qi = (q[:, sl] * scale).astype(jnp.bfloat16)
        ki = kv_scr[:, hh * _DH : (hh + 1) * _DH]
        vi = kv_scr[:, _D + hh * _DH : _D + (hh + 1) * _DH]
        s = jax.lax.dot_general(
            qi, ki, (((1,), (1,)), ((), ())),
            preferred_element_type=jnp.float32,
        )
        es = jnp.exp(s - jnp.max(s, axis=-1, keepdims=True))
        denom = jnp.sum(es, axis=-1, keepdims=True)
        pieces.append(_bmm(es, vi) / denom)
    ctx = jnp.concatenate(pieces, axis=1)
    h1 = _ln(x + _bmm(ctx, wo_ref[0]) + bo_ref[0], l1g_ref[0], l1b_ref[0])
    acc = h1 + b2_ref[0]
    fc = _FF // 2
    for cc in range(2):
        fsl = slice(cc * fc, (cc + 1) * fc)
        g = _gelu(_bmm(h1, w1_ref[0][:, fsl]) + b1_ref[0][:, fsl])
        acc = acc + _bmm(g, w2_ref[0][fsl, :])
    h2 = _ln(acc, l2g_ref[0], l2b_ref[0])
    h_scr[rows_q, :] = h2

    @pl.when(jnp.logical_and(l == _L - 1, half == 0))
    def _out():
        out_ref[...] = h2[0:1, :].reshape(1, 1, _D)


def _pooler_kernel(x_ref, wp_ref, bp_ref, wc_ref, bc_ref, out_ref):
    x = x_ref[...].astype(jnp.float32)
    pooled = jnp.tanh(_mm(x, wp_ref[...]) + bp_ref[...])
    logits = _mm(pooled, wc_ref[...]) + bc_ref[...]
    logits = logits - jnp.max(logits, axis=-1, keepdims=True)
    e = jnp.exp(logits)
    out_ref[...] = e / jnp.sum(e, axis=-1, keepdims=True)


def kernel(params, inputs):
    p = params
    ids = inputs.reshape(-1)

    eraw = pl.pallas_call(
        _gather_kernel,
        grid_spec=pltpu.PrefetchScalarGridSpec(
            num_scalar_prefetch=1,
            grid=(_M // _GC,),
            in_specs=[
                pl.BlockSpec((1, 1, _D), functools.partial(_gather_index, j))
                for j in range(_GC)
            ],
            out_specs=pl.BlockSpec((_GC, _D), lambda i, idx: (i, 0)),
        ),
        out_shape=jax.ShapeDtypeStruct((_M, _D), jnp.bfloat16),
    )(ids, *([p['word_emb'].reshape(-1, 1, _D)] * _GC))

    pos = p['pos_emb'][:_S].astype(jnp.bfloat16)
    tok = p['tok_emb'][0].reshape(1, _D)
    eg = p['emb_ln_g'].reshape(1, _D)
    eb = p['emb_ln_b'].reshape(1, _D)

    def r3(a):
        return a.reshape(_L, 1, a.shape[-1])

    def lspec(shp):
        return pl.BlockSpec(shp, lambda l, t: (l, 0, 0))

    def cnst(shp):
        return pl.BlockSpec(shp, lambda l, t: (0, 0))

    bf = jnp.bfloat16
    cls = pl.pallas_call(
        _mega_kernel,
        grid=(_L, _NT),
        in_specs=[
            pl.BlockSpec(
                (_S, _D), lambda l, t: (jnp.where(l == 0, t // 2, _B - 1), 0)
            ),
            cnst((_S, _D)),
            cnst((1, _D)), cnst((1, _D)), cnst((1, _D)),
            lspec((1, _D, _D)), lspec((1, 1, _D)),
            lspec((1, _D, _D)), lspec((1, 1, _D)),
            lspec((1, _D, _D)), lspec((1, 1, _D)),
            lspec((1, _D, _D)), lspec((1, 1, _D)),
            lspec((1, 1, _D)), lspec((1, 1, _D)),
            lspec((1, _D, _FF)), lspec((1, 1, _FF)),
            lspec((1, _FF, _D)), lspec((1, 1, _D)),
            lspec((1, 1, _D)), lspec((1, 1, _D)),
        ],
        out_specs=pl.BlockSpec(
            (1, 1, _D), lambda l, t: (jnp.where(l == _L - 1, t // 2, 0), 0, 0)
        ),
        out_shape=jax.ShapeDtypeStruct((_B, 1, _D), jnp.float32),
        scratch_shapes=[
            pltpu.VMEM((_M, _D), jnp.float32),
            pltpu.VMEM((_S, 2 * _D), jnp.bfloat16),
        ],
        compiler_params=pltpu.CompilerParams(
            dimension_semantics=("arbitrary", "arbitrary"),
        ),
    )(
        eraw, pos, tok, eg, eb,
        p['Wq'].astype(bf), r3(p['bq']),
        p['Wk'].astype(bf), r3(p['bk']),
        p['Wv'].astype(bf), r3(p['bv']),
        p['Wo'].astype(bf), r3(p['bo']),
        r3(p['ln1_g']), r3(p['ln1_b']),
        p['W1'].astype(bf), r3(p['b1']),
        p['W2'].astype(bf), r3(p['b2']),
        r3(p['ln2_g']), r3(p['ln2_b']),
    )

    x0 = cls.reshape(_B, _D)
    wc = jnp.zeros((_D, 128), jnp.float32).at[:, :3].set(p['Wc'])
    bc = jnp.full((1, 128), -1e30, jnp.float32).at[0, :3].set(p['bc'])
    probs = pl.pallas_call(
        _pooler_kernel,
        out_shape=jax.ShapeDtypeStruct((_B, 128), jnp.float32),
    )(x0, p['Wp'], p['bp'].reshape(1, _D), wc, bc)
    return probs[:, :3]


# speedup vs baseline: 1.5520x; 1.5520x over previous
import functools
import math

import jax
import jax.numpy as jnp
from jax import lax
from jax.experimental import pallas as pl
from jax.experimental.pallas import tpu as pltpu
from jax.experimental.pallas import tpu_sc as plsc

_L = 12
_D = 768
_H = 12
_DH = 64
_FF = 3072
_B = 8
_S = 512
_M = _B * _S
_GC = 32


def _make_sc_gather():
    info = plsc.get_sparse_core_info()
    nw = info.num_cores * info.num_subcores
    b_per_w = _M // nw
    mesh = plsc.VectorSubcoreMesh(core_axis_name="c", subcore_axis_name="s")

    @functools.partial(
        pl.kernel,
        mesh=mesh,
        out_type=jax.ShapeDtypeStruct((_M, _D), jnp.float32),
        scratch_types=[
            pltpu.VMEM((b_per_w,), jnp.int32),
            pltpu.VMEM((b_per_w, _D), jnp.float32),
            pltpu.SemaphoreType.DMA,
        ],
    )
    def k(table_hbm, idx_hbm, out_hbm, idx_v, rows_v, sem):
        wid = lax.axis_index("s") * info.num_cores + lax.axis_index("c")
        base = wid * b_per_w
        pltpu.sync_copy(idx_hbm.at[pl.ds(base, b_per_w)], idx_v)
        pltpu.async_copy(table_hbm.at[idx_v], rows_v, sem).wait()
        pltpu.sync_copy(rows_v, out_hbm.at[pl.ds(base, b_per_w)])

    return k


def _ln(x, g, b):
    m = jnp.mean(x, axis=-1, keepdims=True)
    v = jnp.mean((x - m) ** 2, axis=-1, keepdims=True)
    return (x - m) * jax.lax.rsqrt(v + 1e-12) * g + b


def _mm(x, w):
    return jax.lax.dot_general(
        x, w, (((1,), (0,)), ((), ())), preferred_element_type=jnp.float32
    )


def _bmm(x, w):
    return jax.lax.dot_general(
        x.astype(jnp.bfloat16), w.astype(jnp.bfloat16),
        (((1,), (0,)), ((), ())), preferred_element_type=jnp.float32,
    )


def _embed_kernel(eraw_ref, pos_ref, tok_ref, eg_ref, eb_ref, out_ref):
    pe = pos_ref[...] + tok_ref[...]
    e1 = _ln(eraw_ref[...] + pe, eg_ref[...], eb_ref[...])
    out_ref[...] = _ln(e1 + pe, eg_ref[...], eb_ref[...])


def _layer_kernel(
    l_ref, h_ref,
    wq_ref, bq_ref, wk_ref, bk_ref, wv_ref, bv_ref,
    wo_ref, bo_ref, l1g_ref, l1b_ref,
    w1_ref, b1_ref, w2_ref, b2_ref, l2g_ref, l2b_ref,
    out_ref,
):
    del l_ref
    x = h_ref[...]
    q = _mm(x, wq_ref[0]) + bq_ref[0]
    k = _mm(x, wk_ref[0]) + bk_ref[0]
    v = _mm(x, wv_ref[0]) + bv_ref[0]
    scale = jnp.float32(1.0 / math.sqrt(_DH))
    pieces = []
    for hh in range(_H):
        sl = slice(hh * _DH, (hh + 1) * _DH)
        qi = q[:, sl] * scale
        ki = k[:, sl]
        vi = v[:, sl]
        s = jax.lax.dot_general(
            qi, ki, (((1,), (1,)), ((), ())), preferred_element_type=jnp.float32
        )
        es = jnp.exp(s - jnp.max(s, axis=-1, keepdims=True))
        denom = jnp.sum(es, axis=-1, keepdims=True)
        pieces.append(_mm(es, vi) / denom)
    ctx = jnp.concatenate(pieces, axis=1)
    h1 = _ln(x + _mm(ctx, wo_ref[0]) + bo_ref[0], l1g_ref[0], l1b_ref[0])
    pre = _mm(h1, w1_ref[0]) + b1_ref[0]
    g = 0.5 * pre * (1.0 + jax.lax.erf(pre * jnp.float32(1.0 / math.sqrt(2.0))))
    out_ref[...] = _ln(
        h1 + _mm(g, w2_ref[0]) + b2_ref[0], l2g_ref[0], l2b_ref[0]
    )


def _pooler_kernel(x_ref, wp_ref, bp_ref, wc_ref, bc_ref, out_ref):
    pooled = jnp.tanh(_mm(x_ref[...], wp_ref[...]) + bp_ref[...])
    logits = _mm(pooled, wc_ref[...]) + bc_ref[...]
    logits = logits - jnp.max(logits, axis=-1, keepdims=True)
    e = jnp.exp(logits)
    out_ref[...] = e / jnp.sum(e, axis=-1, keepdims=True)


def kernel(params, inputs):
    p = params
    ids = inputs.reshape(-1)

    eraw = _make_sc_gather()(p['word_emb'], ids)

    pos = p['pos_emb'][:_S]
    tok = p['tok_emb'][0].reshape(1, _D)
    eg = p['emb_ln_g'].reshape(1, _D)
    eb = p['emb_ln_b'].reshape(1, _D)

    h = pl.pallas_call(
        _embed_kernel,
        grid=(_B,),
        in_specs=[
            pl.BlockSpec((_S, _D), lambda b: (b, 0)),
            pl.BlockSpec((_S, _D), lambda b: (0, 0)),
            pl.BlockSpec((1, _D), lambda b: (0, 0)),
            pl.BlockSpec((1, _D), lambda b: (0, 0)),
            pl.BlockSpec((1, _D), lambda b: (0, 0)),
        ],
        out_specs=pl.BlockSpec((_S, _D), lambda b: (b, 0)),
        out_shape=jax.ShapeDtypeStruct((_M, _D), jnp.float32),
    )(eraw, pos, tok, eg, eb)

    def r3(a):
        return a.reshape(_L, 1, a.shape[-1])

    def wspec(shp):
        return pl.BlockSpec(shp, lambda b, lref: (lref[0], 0, 0))

    layer_call = pl.pallas_call(
        _layer_kernel,
        grid_spec=pltpu.PrefetchScalarGridSpec(
            num_scalar_prefetch=1,
            grid=(_B,),
            in_specs=[
                pl.BlockSpec((_S, _D), lambda b, lref: (b, 0)),
                wspec((1, _D, _D)), wspec((1, 1, _D)),
                wspec((1, _D, _D)), wspec((1, 1, _D)),
                wspec((1, _D, _D)), wspec((1, 1, _D)),
                wspec((1, _D, _D)), wspec((1, 1, _D)),
                wspec((1, 1, _D)), wspec((1, 1, _D)),
                wspec((1, _D, _FF)), wspec((1, 1, _FF)),
                wspec((1, _FF, _D)), wspec((1, 1, _D)),
                wspec((1, 1, _D)), wspec((1, 1, _D)),
            ],
            out_specs=pl.BlockSpec((_S, _D), lambda b, lref: (b, 0)),
        ),
        out_shape=jax.ShapeDtypeStruct((_M, _D), jnp.float32),
        compiler_params=pltpu.CompilerParams(
            dimension_semantics=("arbitrary",),
        ),
    )

    wq, wk, wv, wo = p['Wq'], p['Wk'], p['Wv'], p['Wo']
    bq, bk, bv, bo = r3(p['bq']), r3(p['bk']), r3(p['bv']), r3(p['bo'])
    l1g, l1b = r3(p['ln1_g']), r3(p['ln1_b'])
    w1, b1, w2, b2 = p['W1'], r3(p['b1']), p['W2'], r3(p['b2'])
    l2g, l2b = r3(p['ln2_g']), r3(p['ln2_b'])
    for l in range(_L):
        h = layer_call(
            jnp.array([l], jnp.int32), h,
            wq, bq, wk, bk, wv, bv, wo, bo, l1g, l1b,
            w1, b1, w2, b2, l2g, l2b,
        )

    x0 = h.reshape(_B, _S, _D)[:, 0, :]
    wc = jnp.zeros((_D, 128), jnp.float32).at[:, :3].set(p['Wc'])
    bc = jnp.full((1, 128), -1e30, jnp.float32).at[0, :3].set(p['bc'])
    probs = pl.pallas_call(
        _pooler_kernel,
        out_shape=jax.ShapeDtypeStruct((_B, 128), jnp.float32),
    )(x0, p['Wp'], p['bp'].reshape(1, _D), wc, bc)
    return probs[:, :3]


# speedup vs baseline: 1.5525x; 1.0003x over previous
import functools
import math

import jax
import jax.numpy as jnp
from jax import lax
from jax.experimental import pallas as pl
from jax.experimental.pallas import tpu as pltpu
from jax.experimental.pallas import tpu_sc as plsc

_L = 12
_D = 768
_H = 12
_DH = 64
_FF = 3072
_B = 8
_S = 512
_M = _B * _S


def _make_sc_gather():
    info = plsc.get_sparse_core_info()
    nw = info.num_cores * info.num_subcores
    b_per_w = _M // nw
    mesh = plsc.VectorSubcoreMesh(core_axis_name="c", subcore_axis_name="s")

    @functools.partial(
        pl.kernel,
        mesh=mesh,
        out_type=jax.ShapeDtypeStruct((_M, _D), jnp.float32),
        scratch_types=[
            pltpu.VMEM((b_per_w,), jnp.int32),
            pltpu.VMEM((b_per_w, _D), jnp.float32),
            pltpu.SemaphoreType.DMA,
        ],
    )
    def k(table_hbm, idx_hbm, out_hbm, idx_v, rows_v, sem):
        wid = lax.axis_index("s") * info.num_cores + lax.axis_index("c")
        base = wid * b_per_w
        pltpu.sync_copy(idx_hbm.at[pl.ds(base, b_per_w)], idx_v)
        pltpu.async_copy(table_hbm.at[idx_v], rows_v, sem).wait()
        pltpu.sync_copy(rows_v, out_hbm.at[pl.ds(base, b_per_w)])

    return k


def _ln(x, g, b):
    m = jnp.mean(x, axis=-1, keepdims=True)
    v = jnp.mean((x - m) ** 2, axis=-1, keepdims=True)
    return (x - m) * jax.lax.rsqrt(v + 1e-12) * g + b


def _mm(x, w):
    return jax.lax.dot_general(
        x, w, (((1,), (0,)), ((), ())), preferred_element_type=jnp.float32
    )


def _embed_kernel(eraw_ref, pos_ref, tok_ref, eg_ref, eb_ref, out_ref):
    pe = pos_ref[...] + tok_ref[...]
    e1 = _ln(eraw_ref[...] + pe, eg_ref[...], eb_ref[...])
    out_ref[...] = _ln(e1 + pe, eg_ref[...], eb_ref[...])


def _layer_kernel(
    l_ref, h_ref,
    wq_ref, bq_ref, wk_ref, bk_ref, wv_ref, bv_ref,
    wo_ref, bo_ref, l1g_ref, l1b_ref,
    w1_ref, b1_ref, w2_ref, b2_ref, l2g_ref, l2b_ref,
    out_ref,
):
    del l_ref
    x = h_ref[...]
    q = _mm(x, wq_ref[0]) + bq_ref[0]
    k = _mm(x, wk_ref[0]) + bk_ref[0]
    v = _mm(x, wv_ref[0]) + bv_ref[0]
    scale = jnp.float32(1.0 / math.sqrt(_DH))
    pieces = []
    for hh in range(_H):
        sl = slice(hh * _DH, (hh + 1) * _DH)
        qi = q[:, sl] * scale
        ki = k[:, sl]
        vi = v[:, sl]
        s = jax.lax.dot_general(
            qi, ki, (((1,), (1,)), ((), ())), preferred_element_type=jnp.float32
        )
        es = jnp.exp(s - jnp.max(s, axis=-1, keepdims=True))
        denom = jnp.sum(es, axis=-1, keepdims=True)
        pieces.append(_mm(es, vi) / denom)
    ctx = jnp.concatenate(pieces, axis=1)
    h1 = _ln(x + _mm(ctx, wo_ref[0]) + bo_ref[0], l1g_ref[0], l1b_ref[0])
    pre = _mm(h1, w1_ref[0]) + b1_ref[0]
    g = 0.5 * pre * (1.0 + jax.lax.erf(pre * jnp.float32(1.0 / math.sqrt(2.0))))
    out_ref[...] = _ln(
        h1 + _mm(g, w2_ref[0]) + b2_ref[0], l2g_ref[0], l2b_ref[0]
    )


def _pooler_kernel(x_ref, wp_ref, bp_ref, wc_ref, bc_ref, out_ref):
    pooled = jnp.tanh(_mm(x_ref[...], wp_ref[...]) + bp_ref[...])
    logits = _mm(pooled, wc_ref[...]) + bc_ref[...]
    logits = logits - jnp.max(logits, axis=-1, keepdims=True)
    e = jnp.exp(logits)
    out_ref[...] = e / jnp.sum(e, axis=-1, keepdims=True)


def kernel(params, inputs):
    p = params
    ids = inputs.reshape(-1)

    eraw = _make_sc_gather()(p['word_emb'], ids)

    pos = p['pos_emb'][:_S]
    tok = p['tok_emb'][0].reshape(1, _D)
    eg = p['emb_ln_g'].reshape(1, _D)
    eb = p['emb_ln_b'].reshape(1, _D)

    h = pl.pallas_call(
        _embed_kernel,
        grid=(_B,),
        in_specs=[
            pl.BlockSpec((_S, _D), lambda b: (b, 0)),
            pl.BlockSpec((_S, _D), lambda b: (0, 0)),
            pl.BlockSpec((1, _D), lambda b: (0, 0)),
            pl.BlockSpec((1, _D), lambda b: (0, 0)),
            pl.BlockSpec((1, _D), lambda b: (0, 0)),
        ],
        out_specs=pl.BlockSpec((_S, _D), lambda b: (b, 0)),
        out_shape=jax.ShapeDtypeStruct((_M, _D), jnp.float32),
    )(eraw, pos, tok, eg, eb)

    def r3(a):
        return a.reshape(_L, 1, a.shape[-1])

    def wspec(shp):
        return pl.BlockSpec(shp, lambda b, lref: (lref[0], 0, 0))

    layer_call = pl.pallas_call(
        _layer_kernel,
        grid_spec=pltpu.PrefetchScalarGridSpec(
            num_scalar_prefetch=1,
            grid=(_B,),
            in_specs=[
                pl.BlockSpec((_S, _D), lambda b, lref: (b, 0)),
                wspec((1, _D, _D)), wspec((1, 1, _D)),
                wspec((1, _D, _D)), wspec((1, 1, _D)),
                wspec((1, _D, _D)), wspec((1, 1, _D)),
                wspec((1, _D, _D)), wspec((1, 1, _D)),
                wspec((1, 1, _D)), wspec((1, 1, _D)),
                wspec((1, _D, _FF)), wspec((1, 1, _FF)),
                wspec((1, _FF, _D)), wspec((1, 1, _D)),
                wspec((1, 1, _D)), wspec((1, 1, _D)),
            ],
            out_specs=pl.BlockSpec((_S, _D), lambda b, lref: (b, 0)),
        ),
        out_shape=jax.ShapeDtypeStruct((_M, _D), jnp.float32),
        compiler_params=pltpu.CompilerParams(
            dimension_semantics=("arbitrary",),
        ),
    )

    wq, wk, wv, wo = p['Wq'], p['Wk'], p['Wv'], p['Wo']
    bq, bk, bv, bo = r3(p['bq']), r3(p['bk']), r3(p['bv']), r3(p['bo'])
    l1g, l1b = r3(p['ln1_g']), r3(p['ln1_b'])
    w1, b1, w2, b2 = p['W1'], r3(p['b1']), p['W2'], r3(p['b2'])
    l2g, l2b = r3(p['ln2_g']), r3(p['ln2_b'])
    for l in range(_L):
        h = layer_call(
            jnp.array([l], jnp.int32), h,
            wq, bq, wk, bk, wv, bv, wo, bo, l1g, l1b,
            w1, b1, w2, b2, l2g, l2b,
        )

    x0 = h.reshape(_B, _S, _D)[:, 0, :]
    wc = jnp.zeros((_D, 128), jnp.float32).at[:, :3].set(p['Wc'])
    bc = jnp.full((1, 128), -1e30, jnp.float32).at[0, :3].set(p['bc'])
    probs = pl.pallas_call(
        _pooler_kernel,
        out_shape=jax.ShapeDtypeStruct((_B, 128), jnp.float32),
    )(x0, p['Wp'], p['bp'].reshape(1, _D), wc, bc)
    return probs[:, :3]
